# baseline (speedup 1.0000x reference)
import jax
import jax.numpy as jnp
from jax import lax
from jax.experimental import pallas as pl
from jax.experimental.pallas import tpu as pltpu
from jax.experimental.pallas import tpu_sc as plsc

_NC, _NS = 2, 16
_NW = _NC * _NS
_N = 100000
_G = 512
_D = 128

_S = 40000
_ROWS_PER_W = _S // _NW
_CH = 125
_NCH = _ROWS_PER_W // _CH
_D_RING = 2

_BLK = 2000


def _sc_segsum_body(nodes, idx2, partial, shared, nbuf, idxv, zbuf, sem_s, sem_c):
    c = lax.axis_index("c")
    s = lax.axis_index("s")
    wid = c * _NS + s
    base = wid * _ROWS_PER_W
    gpt = _G // _NS

    def stage(j, b):
        pltpu.async_copy(
            nodes.at[pl.ds(base + j * _CH, _CH), :], nbuf.at[b], sem_s.at[b]
        )

    for b in range(_D_RING):
        stage(b, b)
    idx_cp = pltpu.async_copy(
        idx2.at[pl.ds(wid * _NCH, _NCH)], idxv, sem_c.at[0]
    )

    @pl.loop(0, gpt)
    def _(i):
        @pl.loop(0, _D // 16, unroll=8)
        def _(k):
            zbuf[i, pl.ds(k * 16, 16)] = jnp.zeros((16,), jnp.float32)

    pltpu.sync_copy(zbuf, shared.at[pl.ds(s * gpt, gpt)])
    idx_cp.wait()

    plsc.subcore_barrier()

    @pl.loop(0, _NCH // _D_RING)
    def _(g):
        j0 = g * _D_RING
        for b in range(_D_RING):
            pltpu.make_async_copy(
                nodes.at[pl.ds(base, _CH), :], nbuf.at[b], sem_s.at[b]
            ).wait()
            pltpu.async_copy(
                nbuf.at[b], shared.at[idxv.at[j0 + b]], sem_c.at[b], add=True
            )
        for b in range(_D_RING):
            pltpu.make_async_copy(
                nbuf.at[b], shared.at[idxv.at[0]], sem_c.at[b]
            ).wait()

            @pl.when(j0 + _D_RING + b < _NCH)
            def _():
                stage(j0 + _D_RING + b, b)

    plsc.subcore_barrier()
    pltpu.sync_copy(
        shared.at[pl.ds(s * gpt, gpt)], partial.at[c, pl.ds(s * gpt, gpt)]
    )


_W = 256


def _tc_onehot_body(g0r_ref, need2_ref, idx_ref, nodes_ref, out_ref, acc_ref):
    i = pl.program_id(0)
    n_blocks = pl.num_programs(0)
    blk0 = _S // _BLK

    @pl.when(i == 0)
    def _():
        acc_ref[...] = jnp.zeros_like(acc_ref)

    g0r = pl.multiple_of(g0r_ref[i + blk0], 8)
    idx = idx_ref[0, 0, :]
    rel = idx - g0r
    nodes_bf = nodes_ref[...].astype(jnp.bfloat16)
    iota = lax.broadcasted_iota(jnp.int32, (_W, _BLK), 0)

    oh0 = (iota == rel[None, :]).astype(jnp.bfloat16)
    acc_ref[pl.ds(g0r, _W), :] += jnp.dot(
        oh0, nodes_bf, preferred_element_type=jnp.float32
    )

    @pl.when(need2_ref[i + blk0] == 1)
    def _():
        oh1 = (iota + _W == rel[None, :]).astype(jnp.bfloat16)
        acc_ref[pl.ds(g0r + _W, _W), :] += jnp.dot(
            oh1, nodes_bf, preferred_element_type=jnp.float32
        )

    @pl.when(i == n_blocks - 1)
    def _():
        out_ref[...] = acc_ref[: _G, :]


def _combine_body(sc_ref, tc_ref, glob_ref, w_ref, b_ref, out_ref):
    segsum = sc_ref[0] + sc_ref[1] + tc_ref[...]
    d_feat = segsum.shape[1]
    w_top = w_ref[:d_feat, :]
    w_bot = w_ref[d_feat:, :]
    out_ref[...] = (
        jnp.dot(segsum, w_top, preferred_element_type=jnp.float32)
        + jnp.dot(glob_ref[...], w_bot, preferred_element_type=jnp.float32)
        + b_ref[...][None, :]
    )


@jax.jit
def kernel(nodes, edges, receivers, senders, global_latent, node_graph_idx,
           edge_graph_idx, W, b):
    n_graphs, d_global = global_latent.shape
    d_out = W.shape[1]

    idx2 = node_graph_idx.reshape(_N // _CH, _CH)

    mesh = plsc.VectorSubcoreMesh(core_axis_name="c", subcore_axis_name="s")
    sc_segsum = pl.kernel(
        _sc_segsum_body,
        out_type=jax.ShapeDtypeStruct((_NC, _G, _D), jnp.float32),
        mesh=mesh,
        scratch_types=[
            pltpu.VMEM_SHARED((_G, _D), jnp.float32),
            pltpu.VMEM((_D_RING, _CH, _D), jnp.float32),
            pltpu.VMEM((_NCH, _CH), jnp.int32),
            pltpu.VMEM((_G // _NS, _D), jnp.float32),
            pltpu.SemaphoreType.DMA((_D_RING,)),
            pltpu.SemaphoreType.DMA((_D_RING,)),
        ],
        compiler_params=pltpu.CompilerParams(use_tc_tiling_on_sc=False),
    )
    sc_partial = sc_segsum(nodes, idx2)

    n_blocks = (_N - _S) // _BLK
    blk0 = _S // _BLK
    idx_tc = node_graph_idx.reshape(_N // _BLK, 1, _BLK)
    g0r = (node_graph_idx[:: _BLK] // 8) * 8
    gmax = node_graph_idx[_BLK - 1 :: _BLK]
    need2 = (gmax - g0r >= _W).astype(jnp.int32)
    tc_partial = pl.pallas_call(
        _tc_onehot_body,
        grid_spec=pltpu.PrefetchScalarGridSpec(
            num_scalar_prefetch=2,
            grid=(n_blocks,),
            in_specs=[
                pl.BlockSpec((1, 1, _BLK), lambda i, *_: (i + blk0, 0, 0)),
                pl.BlockSpec((_BLK, _D), lambda i, *_: (i + blk0, 0)),
            ],
            out_specs=pl.BlockSpec((_G, _D), lambda i, *_: (0, 0)),
            scratch_shapes=[pltpu.VMEM((2 * _G, _D), jnp.float32)],
        ),
        out_shape=jax.ShapeDtypeStruct((_G, _D), jnp.float32),
    )(g0r, need2, idx_tc, nodes)

    return pl.pallas_call(
        _combine_body,
        out_shape=jax.ShapeDtypeStruct((n_graphs, d_out), jnp.float32),
    )(sc_partial, tc_partial, global_latent, W, b)


# speedup vs baseline: 1.1302x; 1.1302x over previous
import jax
import jax.numpy as jnp
from jax import lax
from jax.experimental import pallas as pl
from jax.experimental.pallas import tpu as pltpu
from jax.experimental.pallas import tpu_sc as plsc

_NC, _NS = 2, 16
_NW = _NC * _NS
_N = 100000
_G = 512
_D = 128

_S = 56000
_ROWS_PER_W = _S // _NW
_CH = 125
_NCH = _ROWS_PER_W // _CH
_D_RING = 2

_BLK = 2000


def _sc_segsum_body(nodes, idx2, partial, shared, nbuf, idxv, zbuf, sem_s, sem_c):
    c = lax.axis_index("c")
    s = lax.axis_index("s")
    wid = c * _NS + s
    base = wid * _ROWS_PER_W
    gpt = _G // _NS

    def stage(j, b):
        pltpu.async_copy(
            nodes.at[pl.ds(base + j * _CH, _CH), :], nbuf.at[b], sem_s.at[b]
        )

    for b in range(_D_RING):
        stage(b, b)
    idx_cp = pltpu.async_copy(
        idx2.at[pl.ds(wid * _NCH, _NCH)], idxv, sem_c.at[0]
    )

    @pl.loop(0, gpt)
    def _(i):
        @pl.loop(0, _D // 16, unroll=8)
        def _(k):
            zbuf[i, pl.ds(k * 16, 16)] = jnp.zeros((16,), jnp.float32)

    pltpu.sync_copy(zbuf, shared.at[pl.ds(s * gpt, gpt)])
    idx_cp.wait()

    plsc.subcore_barrier()

    @pl.loop(0, _NCH // _D_RING)
    def _(g):
        j0 = g * _D_RING
        for b in range(_D_RING):
            pltpu.make_async_copy(
                nodes.at[pl.ds(base, _CH), :], nbuf.at[b], sem_s.at[b]
            ).wait()
            pltpu.async_copy(
                nbuf.at[b], shared.at[idxv.at[j0 + b]], sem_c.at[b], add=True
            )
        for b in range(_D_RING):
            pltpu.make_async_copy(
                nbuf.at[b], shared.at[idxv.at[0]], sem_c.at[b]
            ).wait()

            @pl.when(j0 + _D_RING + b < _NCH)
            def _():
                stage(j0 + _D_RING + b, b)

    plsc.subcore_barrier()
    pltpu.sync_copy(
        shared.at[pl.ds(s * gpt, gpt)], partial.at[c, pl.ds(s * gpt, gpt)]
    )


_W = 256


def _tc_onehot_body(g0r_ref, need2_ref, idx_ref, nodes_ref, out_ref, acc_ref):
    i = pl.program_id(0)
    n_blocks = pl.num_programs(0)
    blk0 = _S // _BLK

    @pl.when(i == 0)
    def _():
        acc_ref[...] = jnp.zeros_like(acc_ref)

    g0r = pl.multiple_of(g0r_ref[i + blk0], 8)
    idx = idx_ref[0, 0, :]
    rel = idx - g0r
    nodes_bf = nodes_ref[...].astype(jnp.bfloat16)
    iota = lax.broadcasted_iota(jnp.int32, (_W, _BLK), 0)

    oh0 = (iota == rel[None, :]).astype(jnp.bfloat16)
    acc_ref[pl.ds(g0r, _W), :] += jnp.dot(
        oh0, nodes_bf, preferred_element_type=jnp.float32
    )

    @pl.when(need2_ref[i + blk0] == 1)
    def _():
        oh1 = (iota + _W == rel[None, :]).astype(jnp.bfloat16)
        acc_ref[pl.ds(g0r + _W, _W), :] += jnp.dot(
            oh1, nodes_bf, preferred_element_type=jnp.float32
        )

    @pl.when(i == n_blocks - 1)
    def _():
        out_ref[...] = acc_ref[: _G, :]


def _combine_body(sc_ref, tc_ref, glob_ref, w_ref, b_ref, out_ref):
    segsum = sc_ref[0] + sc_ref[1] + tc_ref[...]
    d_feat = segsum.shape[1]
    w_top = w_ref[:d_feat, :]
    w_bot = w_ref[d_feat:, :]
    out_ref[...] = (
        jnp.dot(segsum, w_top, preferred_element_type=jnp.float32)
        + jnp.dot(glob_ref[...], w_bot, preferred_element_type=jnp.float32)
        + b_ref[...][None, :]
    )


@jax.jit
def kernel(nodes, edges, receivers, senders, global_latent, node_graph_idx,
           edge_graph_idx, W, b):
    n_graphs, d_global = global_latent.shape
    d_out = W.shape[1]

    idx2 = node_graph_idx.reshape(_N // _CH, _CH)

    mesh = plsc.VectorSubcoreMesh(core_axis_name="c", subcore_axis_name="s")
    sc_segsum = pl.kernel(
        _sc_segsum_body,
        out_type=jax.ShapeDtypeStruct((_NC, _G, _D), jnp.float32),
        mesh=mesh,
        scratch_types=[
            pltpu.VMEM_SHARED((_G, _D), jnp.float32),
            pltpu.VMEM((_D_RING, _CH, _D), jnp.float32),
            pltpu.VMEM((_NCH, _CH), jnp.int32),
            pltpu.VMEM((_G // _NS, _D), jnp.float32),
            pltpu.SemaphoreType.DMA((_D_RING,)),
            pltpu.SemaphoreType.DMA((_D_RING,)),
        ],
        compiler_params=pltpu.CompilerParams(use_tc_tiling_on_sc=False),
    )
    sc_partial = sc_segsum(nodes, idx2)

    n_blocks = (_N - _S) // _BLK
    blk0 = _S // _BLK
    idx_tc = node_graph_idx.reshape(_N // _BLK, 1, _BLK)
    g0r = (node_graph_idx[:: _BLK] // 8) * 8
    gmax = node_graph_idx[_BLK - 1 :: _BLK]
    need2 = (gmax - g0r >= _W).astype(jnp.int32)
    tc_partial = pl.pallas_call(
        _tc_onehot_body,
        grid_spec=pltpu.PrefetchScalarGridSpec(
            num_scalar_prefetch=2,
            grid=(n_blocks,),
            in_specs=[
                pl.BlockSpec((1, 1, _BLK), lambda i, *_: (i + blk0, 0, 0)),
                pl.BlockSpec((_BLK, _D), lambda i, *_: (i + blk0, 0)),
            ],
            out_specs=pl.BlockSpec((_G, _D), lambda i, *_: (0, 0)),
            scratch_shapes=[pltpu.VMEM((2 * _G, _D), jnp.float32)],
        ),
        out_shape=jax.ShapeDtypeStruct((_G, _D), jnp.float32),
    )(g0r, need2, idx_tc, nodes)

    return pl.pallas_call(
        _combine_body,
        out_shape=jax.ShapeDtypeStruct((n_graphs, d_out), jnp.float32),
    )(sc_partial, tc_partial, global_latent, W, b)
